# baseline (speedup 1.0000x reference)
import functools

import jax
import jax.numpy as jnp
from jax import lax
from jax.experimental import pallas as pl
from jax.experimental.pallas import tpu as pltpu
from jax.experimental.pallas import tpu_sc as plsc

F = 26
B = 16384
V = 100000
D = 32
OUT_W = F * D + 1

BK = 512
RSTEPS = (V + BK - 1) // BK


def _reformat_body(*refs):
    sel = refs[0]
    emb = refs[1]
    ins = refs[2:2 + F]
    outs = refs[2 + F:]
    dn = (((1,), (1,)), ((), ()))
    dn2 = (((1,), (0,)), ((), ()))
    for f in range(F):
        x = ins[f][...]
        acc = None
        for b in range(4):
            piece = jax.lax.dot_general(
                sel[b], x, dn, preferred_element_type=jnp.float32)
            placed = jax.lax.dot_general(
                piece, emb[b], dn2, preferred_element_type=jnp.float32)
            acc = placed if acc is None else acc + placed
        outs[f][...] = acc


_reformat = pl.pallas_call(
    _reformat_body,
    grid=(RSTEPS,),
    in_specs=[pl.BlockSpec((4, BK // 4, BK), lambda i: (0, 0, 0)),
              pl.BlockSpec((4, 32, 128), lambda i: (0, 0, 0))]
    + [pl.BlockSpec((32, BK), lambda i: (0, i))] * F,
    out_specs=[pl.BlockSpec((BK // 4, 128), lambda i: (i, 0))] * F,
    out_shape=[jax.ShapeDtypeStruct((V * D // 128, 128), jnp.float32)] * F,
)


def _selection_matrices():
    b_i = jax.lax.broadcasted_iota(jnp.int32, (4, BK // 4, BK), 0)
    r_i = jax.lax.broadcasted_iota(jnp.int32, (4, BK // 4, BK), 1)
    c_i = jax.lax.broadcasted_iota(jnp.int32, (4, BK // 4, BK), 2)
    return (c_i == 4 * r_i + b_i).astype(jnp.float32)


def _embed_matrices():
    b_i = jax.lax.broadcasted_iota(jnp.int32, (4, 32, 128), 0)
    d_i = jax.lax.broadcasted_iota(jnp.int32, (4, 32, 128), 1)
    c_i = jax.lax.broadcasted_iota(jnp.int32, (4, 32, 128), 2)
    return (c_i == 32 * b_i + d_i).astype(jnp.float32)

NC = 2
NS = 16
NW = NC * NS
BPW = B // NW
CH = 128
NCHUNK = BPW // CH


def _make_kernel():
    mesh = plsc.VectorSubcoreMesh(core_axis_name="c", subcore_axis_name="s")

    @functools.partial(
        pl.kernel,
        mesh=mesh,
        out_type=jax.ShapeDtypeStruct((B, OUT_W), jnp.float32),
        scratch_types=[
            pltpu.VMEM((F, BPW), jnp.int32),
            pltpu.VMEM((F * CH, D), jnp.float32),
            pltpu.VMEM((CH, 1), jnp.float32),
            pltpu.SemaphoreType.DMA,
            pltpu.SemaphoreType.DMA,
        ],
        compiler_params=pltpu.CompilerParams(use_tc_tiling_on_sc=False),
    )
    def k(idx_hbm, cum_hbm, *rest):
        tables = rest[:F]
        out_hbm = rest[F]
        idx_v, gat_v, cum_v, gsem, wsem = rest[F + 1:]

        wid = lax.axis_index("s") * NC + lax.axis_index("c")
        base = wid * BPW

        pltpu.sync_copy(idx_hbm.at[:, pl.ds(base, BPW)], idx_v)

        def chunk(ci, _):
            coff = pl.multiple_of(ci * CH, CH)
            cbase = base + coff
            gathers = []
            for f in range(F):
                gathers.append(pltpu.async_copy(
                    tables[f].at[idx_v.at[f, pl.ds(coff, CH)]],
                    gat_v.at[pl.ds(f * CH, CH), :],
                    gsem))
            pltpu.sync_copy(cum_hbm.at[pl.ds(cbase, CH), :], cum_v)
            for g in gathers:
                g.wait()
            writes = []
            for f in range(F):
                writes.append(pltpu.async_copy(
                    gat_v.at[pl.ds(f * CH, CH), :],
                    out_hbm.at[pl.ds(cbase, CH), pl.ds(f * D, D)],
                    wsem))
            writes.append(pltpu.async_copy(
                cum_v, out_hbm.at[pl.ds(cbase, CH), pl.ds(F * D, 1)], wsem))
            for w in writes:
                w.wait()
            return 0

        lax.fori_loop(0, NCHUNK, chunk, 0)

    return k


_gather_concat = jax.jit(_make_kernel())


def kernel(idx_0, idx_1, idx_2, idx_3, idx_4, idx_5, idx_6, idx_7, idx_8,
           idx_9, idx_10, idx_11, idx_12, idx_13, idx_14, idx_15, idx_16,
           idx_17, idx_18, idx_19, idx_20, idx_21, idx_22, idx_23, idx_24,
           idx_25, cumul_dist_km,
           table_0, table_1, table_2, table_3, table_4, table_5, table_6,
           table_7, table_8, table_9, table_10, table_11, table_12, table_13,
           table_14, table_15, table_16, table_17, table_18, table_19,
           table_20, table_21, table_22, table_23, table_24, table_25):
    idxs = [idx_0, idx_1, idx_2, idx_3, idx_4, idx_5, idx_6, idx_7, idx_8,
            idx_9, idx_10, idx_11, idx_12, idx_13, idx_14, idx_15, idx_16,
            idx_17, idx_18, idx_19, idx_20, idx_21, idx_22, idx_23, idx_24,
            idx_25]
    tables = [table_0, table_1, table_2, table_3, table_4, table_5, table_6,
              table_7, table_8, table_9, table_10, table_11, table_12,
              table_13, table_14, table_15, table_16, table_17, table_18,
              table_19, table_20, table_21, table_22, table_23, table_24,
              table_25]
    idx_all = jnp.stack([i.reshape(-1).astype(jnp.int32) for i in idxs])
    cum = cumul_dist_km.reshape(B, 1).astype(jnp.float32)
    lin = _reformat(_selection_matrices(), _embed_matrices(),
                    *[t.T for t in tables])
    tabs = [lt.reshape(V, D) for lt in lin]
    return _gather_concat(idx_all, cum, *tabs)


# speedup vs baseline: 3.8279x; 3.8279x over previous
import functools

import jax
import jax.numpy as jnp
from jax import lax
from jax.experimental import pallas as pl
from jax.experimental.pallas import tpu as pltpu
from jax.experimental.pallas import tpu_sc as plsc

F = 26
B = 16384
V = 100000
D = 32
OUT_W = F * D + 1

BK = 512
RSTEPS = (V + BK - 1) // BK


def _reformat_body(*refs):
    sel = refs[0]
    ins = refs[1:1 + F]
    outs = refs[1 + F:]
    dn = (((1,), (1,)), ((), ()))
    x_all = jnp.concatenate([ins[f][...] for f in range(F)],
                            axis=0).astype(jnp.bfloat16)
    piece = jax.lax.dot_general(
        sel[...], x_all, dn, preferred_element_type=jnp.float32)
    for f in range(F):
        blk = jax.lax.slice(piece, (0, 32 * f), (BK, 32 * f + 32))
        outs[f][...] = jnp.concatenate(
            [jax.lax.slice(blk, (128 * b, 0), (128 * b + 128, 32))
             for b in range(4)], axis=1)


_reformat = pl.pallas_call(
    _reformat_body,
    grid=(RSTEPS,),
    in_specs=[pl.BlockSpec((BK, BK), lambda i: (0, 0))]
    + [pl.BlockSpec((32, BK), lambda i: (0, i))] * F,
    out_specs=[pl.BlockSpec((BK // 4, 128), lambda i: (i, 0))] * F,
    out_shape=[jax.ShapeDtypeStruct((V * D // 128, 128), jnp.float32)] * F,
)


def _selection_matrix():
    r_i = jax.lax.broadcasted_iota(jnp.int32, (BK, BK), 0)
    c_i = jax.lax.broadcasted_iota(jnp.int32, (BK, BK), 1)
    a_i = r_i % 128
    b_i = r_i // 128
    return (c_i == 4 * a_i + b_i).astype(jnp.bfloat16)

NC = 2
NS = 16
NW = NC * NS
BPW = B // NW
CH = 128
NCHUNK = BPW // CH


def _make_kernel():
    mesh = plsc.VectorSubcoreMesh(core_axis_name="c", subcore_axis_name="s")

    @functools.partial(
        pl.kernel,
        mesh=mesh,
        out_type=jax.ShapeDtypeStruct((B, OUT_W), jnp.float32),
        scratch_types=[
            pltpu.VMEM((F, BPW), jnp.int32),
            pltpu.VMEM((F * CH, D), jnp.float32),
            pltpu.VMEM((CH, 1), jnp.float32),
            pltpu.SemaphoreType.DMA,
            pltpu.SemaphoreType.DMA,
        ],
        compiler_params=pltpu.CompilerParams(use_tc_tiling_on_sc=False),
    )
    def k(idx_hbm, cum_hbm, *rest):
        tables = rest[:F]
        out_hbm = rest[F]
        idx_v, gat_v, cum_v, gsem, wsem = rest[F + 1:]

        wid = lax.axis_index("s") * NC + lax.axis_index("c")
        base = wid * BPW

        pltpu.sync_copy(idx_hbm.at[:, pl.ds(base, BPW)], idx_v)

        def chunk(ci, _):
            coff = pl.multiple_of(ci * CH, CH)
            cbase = base + coff
            gathers = []
            for f in range(F):
                gathers.append(pltpu.async_copy(
                    tables[f].at[idx_v.at[f, pl.ds(coff, CH)]],
                    gat_v.at[pl.ds(f * CH, CH), :],
                    gsem))
            pltpu.sync_copy(cum_hbm.at[pl.ds(cbase, CH), :], cum_v)
            for g in gathers:
                g.wait()
            writes = []
            for f in range(F):
                writes.append(pltpu.async_copy(
                    gat_v.at[pl.ds(f * CH, CH), :],
                    out_hbm.at[pl.ds(cbase, CH), pl.ds(f * D, D)],
                    wsem))
            writes.append(pltpu.async_copy(
                cum_v, out_hbm.at[pl.ds(cbase, CH), pl.ds(F * D, 1)], wsem))
            for w in writes:
                w.wait()
            return 0

        lax.fori_loop(0, NCHUNK, chunk, 0)

    return k


_gather_concat = jax.jit(_make_kernel())


def kernel(idx_0, idx_1, idx_2, idx_3, idx_4, idx_5, idx_6, idx_7, idx_8,
           idx_9, idx_10, idx_11, idx_12, idx_13, idx_14, idx_15, idx_16,
           idx_17, idx_18, idx_19, idx_20, idx_21, idx_22, idx_23, idx_24,
           idx_25, cumul_dist_km,
           table_0, table_1, table_2, table_3, table_4, table_5, table_6,
           table_7, table_8, table_9, table_10, table_11, table_12, table_13,
           table_14, table_15, table_16, table_17, table_18, table_19,
           table_20, table_21, table_22, table_23, table_24, table_25):
    idxs = [idx_0, idx_1, idx_2, idx_3, idx_4, idx_5, idx_6, idx_7, idx_8,
            idx_9, idx_10, idx_11, idx_12, idx_13, idx_14, idx_15, idx_16,
            idx_17, idx_18, idx_19, idx_20, idx_21, idx_22, idx_23, idx_24,
            idx_25]
    tables = [table_0, table_1, table_2, table_3, table_4, table_5, table_6,
              table_7, table_8, table_9, table_10, table_11, table_12,
              table_13, table_14, table_15, table_16, table_17, table_18,
              table_19, table_20, table_21, table_22, table_23, table_24,
              table_25]
    idx_all = jnp.stack([i.reshape(-1).astype(jnp.int32) for i in idxs])
    cum = cumul_dist_km.reshape(B, 1).astype(jnp.float32)
    lin = _reformat(_selection_matrix(), *[t.T for t in tables])
    tabs = [lt.reshape(V, D) for lt in lin]
    return _gather_concat(idx_all, cum, *tabs)


# speedup vs baseline: 5.0446x; 1.3179x over previous
import functools

import jax
import jax.numpy as jnp
from jax import lax
from jax.experimental import pallas as pl
from jax.experimental.pallas import tpu as pltpu
from jax.experimental.pallas import tpu_sc as plsc

F = 26
B = 16384
V = 100000
D = 32
OUT_W = F * D + 1

BK = 512
RSTEPS = (V + BK - 1) // BK


NG = 7


def _reformat_body(*refs):
    sel = refs[0]
    ins = refs[1:1 + F]
    outs = refs[1 + F:]
    dn = (((1,), (1,)), ((), ()))
    x_all = jnp.concatenate(
        [ins[f][...] for f in range(F)]
        + [jnp.zeros((32 * (4 * NG - F), BK), jnp.float32)],
        axis=0).astype(jnp.bfloat16)
    piece = jax.lax.dot_general(
        sel[...], x_all, dn, preferred_element_type=jnp.float32)
    for j in range(NG):
        outs[j][...] = jax.lax.slice(
            piece, (0, 128 * j), (BK, 128 * j + 128))


_reformat = pl.pallas_call(
    _reformat_body,
    grid=(RSTEPS,),
    in_specs=[pl.BlockSpec((BK, BK), lambda i: (0, 0))]
    + [pl.BlockSpec((32, BK), lambda i: (0, i))] * F,
    out_specs=[pl.BlockSpec((BK, 128), lambda i: (i, 0))] * NG,
    out_shape=[jax.ShapeDtypeStruct((RSTEPS * BK, 128), jnp.float32)] * NG,
)


def _selection_matrix():
    r_i = jax.lax.broadcasted_iota(jnp.int32, (BK, BK), 0)
    c_i = jax.lax.broadcasted_iota(jnp.int32, (BK, BK), 1)
    a_i = r_i % 128
    b_i = r_i // 128
    return (c_i == 4 * a_i + b_i).astype(jnp.bfloat16)

NC = 2
NS = 16
NW = NC * NS
BPW = B // NW
CH = 128
NCHUNK = BPW // CH


def _make_kernel():
    mesh = plsc.VectorSubcoreMesh(core_axis_name="c", subcore_axis_name="s")

    @functools.partial(
        pl.kernel,
        mesh=mesh,
        out_type=jax.ShapeDtypeStruct((B, OUT_W), jnp.float32),
        scratch_types=[
            pltpu.VMEM((F, BPW), jnp.int32),
            pltpu.VMEM((F * CH, D), jnp.float32),
            pltpu.VMEM((CH, 1), jnp.float32),
            pltpu.SemaphoreType.DMA,
            pltpu.SemaphoreType.DMA,
        ],
        compiler_params=pltpu.CompilerParams(use_tc_tiling_on_sc=False),
    )
    def k(idx_hbm, cum_hbm, *rest):
        tables = rest[:NG]
        out_hbm = rest[NG]
        idx_v, gat_v, cum_v, gsem, wsem = rest[NG + 1:]

        wid = lax.axis_index("s") * NC + lax.axis_index("c")
        base = wid * BPW

        pltpu.sync_copy(idx_hbm.at[:, pl.ds(base, BPW)], idx_v)

        def chunk(ci, _):
            coff = pl.multiple_of(ci * CH, CH)
            cbase = base + coff
            gathers = []
            for f in range(F):
                gathers.append(pltpu.async_copy(
                    tables[f // 4].at[idx_v.at[f, pl.ds(coff, CH)]],
                    gat_v.at[pl.ds(f * CH, CH), :],
                    gsem))
            pltpu.sync_copy(cum_hbm.at[pl.ds(cbase, CH), :], cum_v)
            for g in gathers:
                g.wait()
            writes = []
            for f in range(F):
                writes.append(pltpu.async_copy(
                    gat_v.at[pl.ds(f * CH, CH), :],
                    out_hbm.at[pl.ds(cbase, CH), pl.ds(f * D, D)],
                    wsem))
            writes.append(pltpu.async_copy(
                cum_v, out_hbm.at[pl.ds(cbase, CH), pl.ds(F * D, 1)], wsem))
            for w in writes:
                w.wait()
            return 0

        lax.fori_loop(0, NCHUNK, chunk, 0)

    return k


_gather_concat = jax.jit(_make_kernel())


def kernel(idx_0, idx_1, idx_2, idx_3, idx_4, idx_5, idx_6, idx_7, idx_8,
           idx_9, idx_10, idx_11, idx_12, idx_13, idx_14, idx_15, idx_16,
           idx_17, idx_18, idx_19, idx_20, idx_21, idx_22, idx_23, idx_24,
           idx_25, cumul_dist_km,
           table_0, table_1, table_2, table_3, table_4, table_5, table_6,
           table_7, table_8, table_9, table_10, table_11, table_12, table_13,
           table_14, table_15, table_16, table_17, table_18, table_19,
           table_20, table_21, table_22, table_23, table_24, table_25):
    idxs = [idx_0, idx_1, idx_2, idx_3, idx_4, idx_5, idx_6, idx_7, idx_8,
            idx_9, idx_10, idx_11, idx_12, idx_13, idx_14, idx_15, idx_16,
            idx_17, idx_18, idx_19, idx_20, idx_21, idx_22, idx_23, idx_24,
            idx_25]
    tables = [table_0, table_1, table_2, table_3, table_4, table_5, table_6,
              table_7, table_8, table_9, table_10, table_11, table_12,
              table_13, table_14, table_15, table_16, table_17, table_18,
              table_19, table_20, table_21, table_22, table_23, table_24,
              table_25]
    v = jnp.stack([i.reshape(-1).astype(jnp.int32) for i in idxs])
    g = (jnp.arange(F, dtype=jnp.int32) % 4)[:, None]
    idx_all = 2048 * (v >> 9) + 512 * (v & 3) + 4 * ((v >> 2) & 127) + g
    cum = cumul_dist_km.reshape(B, 1).astype(jnp.float32)
    lin = _reformat(_selection_matrix(), *[t.T for t in tables])
    gtabs = [lt.reshape(RSTEPS * BK * 4, D) for lt in lin]
    return _gather_concat(idx_all, cum, *gtabs)
